# baseline (speedup 1.0000x reference)
import functools

import jax
import jax.numpy as jnp
from jax import lax
from jax.experimental import pallas as pl
from jax.experimental.pallas import tpu as pltpu
from jax.experimental.pallas import tpu_sc as plsc

SEP = 102
DIM = 128
NC, NS = 2, 16
NW = NC * NS
W = 128
LANES = 16


def _prep_body(seq, ids_ref, tt_ref, pos_ref, cidx_ref, add_ref):
    ids = ids_ref[...]
    sep = (ids == SEP).astype(jnp.int32)
    c = sep
    sh = 1
    while sh < seq:
        z = jnp.zeros((ids.shape[0], sh), jnp.int32)
        c = c + jnp.concatenate([z, c[:, : seq - sh]], axis=1)
        sh *= 2
    seg = jnp.minimum(c - sep, 2)
    col = lax.broadcasted_iota(jnp.int32, ids.shape, 1)
    cidx_ref[...] = col + seq * seg
    pos = pos_ref[:seq, :]
    add_ref[:seq, :] = pos + tt_ref[0:1, :]
    add_ref[seq : 2 * seq, :] = pos + tt_ref[1:2, :]
    add_ref[2 * seq : 3 * seq, :] = pos


def _gather_body(nwin, table_hbm, addend_hbm, tid_hbm, cidx_hbm, out_hbm,
                 tid_v, cid_v, row_v0, row_v1, add_v0, add_v1, ost0, ost1,
                 sem_t0, sem_t1, sem_a0, sem_a1, sem_w0, sem_w1):
    row_v = (row_v0, row_v1)
    add_v = (add_v0, add_v1)
    ost = (ost0, ost1)
    sem_t = (sem_t0, sem_t1)
    sem_a = (sem_a0, sem_a1)
    sem_w = (sem_w0, sem_w1)

    wid = lax.axis_index("s") * NC + lax.axis_index("c")
    wbase = wid * (nwin * W)
    pltpu.sync_copy(tid_hbm.at[pl.ds(wbase, nwin * W)], tid_v)
    pltpu.sync_copy(cidx_hbm.at[pl.ds(wbase, nwin * W)], cid_v)

    def issue_gathers(p, ww):
        sl = pl.ds(ww * W, W)
        pltpu.async_copy(table_hbm.at[tid_v.at[sl]], row_v[p], sem_t[p])
        pltpu.async_copy(addend_hbm.at[cid_v.at[sl]], add_v[p], sem_a[p])

    for p in range(2):
        issue_gathers(p, p)

    @pl.loop(0, nwin, step=2)
    def _(w):
        for p in range(2):
            ww = w + p
            pltpu.make_async_copy(
                table_hbm.at[tid_v.at[pl.ds(0, W)]], row_v[p], sem_t[p]).wait()
            pltpu.make_async_copy(
                addend_hbm.at[cid_v.at[pl.ds(0, W)]], add_v[p], sem_a[p]).wait()

            @pl.when(ww >= 2)
            def _():
                pltpu.make_async_copy(
                    ost[p], out_hbm.at[pl.ds(0, W)], sem_w[p]).wait()

            @pl.loop(0, W)
            def _(r):
                for ch in range(DIM // LANES):
                    slc = (pl.ds(r, 1), pl.ds(ch * LANES, LANES))
                    ost[p].at[slc][...] = (
                        row_v[p].at[slc][...] + add_v[p].at[slc][...])

            pltpu.async_copy(ost[p], out_hbm.at[pl.ds(wbase + ww * W, W)],
                             sem_w[p])

            @pl.when(ww + 2 < nwin)
            def _():
                issue_gathers(p, ww + 2)

    for p in range(2):
        pltpu.make_async_copy(ost[p], out_hbm.at[pl.ds(0, W)], sem_w[p]).wait()


def kernel(token_ids, token_emb_table, token_type_emb_table, full_position_emb_table):
    batch, seq = token_ids.shape
    token_ids = token_ids.astype(jnp.int32)

    cidx, addend = pl.pallas_call(
        functools.partial(_prep_body, seq),
        out_shape=[
            jax.ShapeDtypeStruct((batch, seq), jnp.int32),
            jax.ShapeDtypeStruct((3 * seq, DIM), jnp.float32),
        ],
    )(token_ids, token_type_emb_table, full_position_emb_table)

    total = batch * seq
    chunk = NW * W * 2
    padded = ((total + chunk - 1) // chunk) * chunk
    tid_flat = token_ids.reshape(-1)
    cid_flat = cidx.reshape(-1)
    if padded != total:
        pad = padded - total
        tid_flat = jnp.pad(tid_flat, (0, pad))
        cid_flat = jnp.pad(cid_flat, (0, pad))
    nwin = padded // (NW * W)

    mesh = plsc.VectorSubcoreMesh(core_axis_name="c", subcore_axis_name="s")
    out = pl.kernel(
        functools.partial(_gather_body, nwin),
        out_type=jax.ShapeDtypeStruct((padded, DIM), jnp.float32),
        mesh=mesh,
        scratch_types=[
            pltpu.VMEM((nwin * W,), jnp.int32),
            pltpu.VMEM((nwin * W,), jnp.int32),
            pltpu.VMEM((W, DIM), jnp.float32),
            pltpu.VMEM((W, DIM), jnp.float32),
            pltpu.VMEM((W, DIM), jnp.float32),
            pltpu.VMEM((W, DIM), jnp.float32),
            pltpu.VMEM((W, DIM), jnp.float32),
            pltpu.VMEM((W, DIM), jnp.float32),
            pltpu.SemaphoreType.DMA,
            pltpu.SemaphoreType.DMA,
            pltpu.SemaphoreType.DMA,
            pltpu.SemaphoreType.DMA,
            pltpu.SemaphoreType.DMA,
            pltpu.SemaphoreType.DMA,
        ],
    )(token_emb_table, addend, tid_flat, cid_flat)
    if padded != total:
        out = out[:total]
    return out.reshape(batch, seq, DIM)
